# baseline (speedup 1.0000x reference)
import jax
import jax.numpy as jnp
from jax import lax
from jax.experimental import pallas as pl


def _body(el_ref, emb_ref, w2_ref, out_ref):
    R, T = el_ref.shape
    _, V, D2 = w2_ref.shape
    H = T // 2
    el = el_ref[...]
    mf = (el != 0).astype(jnp.bfloat16)
    r_i = lax.broadcasted_iota(jnp.int32, (T, T), 0)
    c_i = lax.broadcasted_iota(jnp.int32, (T, T), 1)
    tri = (r_i <= c_i).astype(jnp.bfloat16)
    cum = jnp.dot(mf, tri, preferred_element_type=jnp.float32)
    pos = (cum * mf.astype(jnp.float32)).astype(jnp.bfloat16)
    s_r = lax.broadcasted_iota(jnp.int32, (T, H), 0)
    s_c = lax.broadcasted_iota(jnp.int32, (T, H), 1)
    se = (s_r == 2 * s_c).astype(jnp.bfloat16)
    so = (s_r == 2 * s_c + 1).astype(jnp.bfloat16)
    pos_e = jnp.dot(pos, se, preferred_element_type=jnp.float32).astype(jnp.bfloat16)
    pos_o = jnp.dot(pos, so, preferred_element_type=jnp.float32).astype(jnp.bfloat16)
    vi = lax.broadcasted_iota(jnp.int32, (1, 1, V), 2).astype(jnp.bfloat16)
    one = jnp.bfloat16(1)
    zero = jnp.bfloat16(0)
    oh_e = jnp.where(pos_e[:, :, None] == vi, one, zero).reshape(R * H, V)
    oh_o = jnp.where(pos_o[:, :, None] == vi, one, zero).reshape(R * H, V)
    pe = (jnp.dot(oh_e, w2_ref[0], preferred_element_type=jnp.float32)
          + jnp.dot(oh_o, w2_ref[1], preferred_element_type=jnp.float32))
    out_ref[...] = emb_ref[...] + pe.reshape(R, H, D2)


def kernel(entity_embeds, entity_list, W):
    B, T, D = entity_embeds.shape
    V = W.shape[0]
    H = T // 2
    R = min(64, B)
    wb = W.astype(jnp.bfloat16)
    z = jnp.zeros_like(wb)
    w2 = jnp.stack([jnp.concatenate([wb, z], axis=1),
                    jnp.concatenate([z, wb], axis=1)])
    emb2 = entity_embeds.reshape(B, H, 2 * D)
    out = pl.pallas_call(
        _body,
        grid=(B // R,),
        in_specs=[
            pl.BlockSpec((R, T), lambda i: (i, 0)),
            pl.BlockSpec((R, H, 2 * D), lambda i: (i, 0, 0)),
            pl.BlockSpec((2, V, 2 * D), lambda i: (0, 0, 0)),
        ],
        out_specs=pl.BlockSpec((R, H, 2 * D), lambda i: (i, 0, 0)),
        out_shape=jax.ShapeDtypeStruct((B, H, 2 * D), jnp.float32),
    )(entity_list, emb2, w2)
    return out.reshape(B, T, D)
